# baseline (speedup 1.0000x reference)
import functools

import jax
import jax.numpy as jnp
from jax import lax
from jax.experimental import pallas as pl
from jax.experimental.pallas import tpu as pltpu
from jax.experimental.pallas import tpu_sc as plsc

N = 4096
E = 131072
C_IN = 256
H = 4
CH = 64
ALPHA = 0.2
NEG = -9e15

NUM_CORES = 2
NUM_SUBCORES = 16
NW = NUM_CORES * NUM_SUBCORES
ROWS_PER_TILE = N // NW
WORDS = N // 32
TILE_WORDS = ROWS_PER_TILE * WORDS
ECH = 16384
NCH = E // ECH


def _bitmask_body(src_hbm, dst_hbm, bm_hbm,
                  src_a, dst_a, src_b, dst_b, bm, sem_a, sem_b):
    cid = lax.axis_index("c")
    sid = lax.axis_index("s")
    wid = cid * NUM_SUBCORES + sid
    lo = wid * ROWS_PER_TILE

    z16 = jnp.zeros((16,), jnp.int32)

    def zf(i, carry):
        bm[pl.ds(i * 16, 16)] = z16
        return carry
    lax.fori_loop(0, TILE_WORDS // 16, zf, 0)

    bufs = [(src_a, dst_a, sem_a), (src_b, dst_b, sem_b)]
    pltpu.make_async_copy(src_hbm.at[pl.ds(0, ECH)], src_a, sem_a).start()
    pltpu.make_async_copy(dst_hbm.at[pl.ds(0, ECH)], dst_a, sem_a).start()

    for g in range(NCH):
        sv, dv, sem = bufs[g % 2]
        pltpu.make_async_copy(src_hbm.at[pl.ds(g * ECH, ECH)], sv, sem).wait()
        pltpu.make_async_copy(dst_hbm.at[pl.ds(g * ECH, ECH)], dv, sem).wait()
        if g + 1 < NCH:
            nsv, ndv, nsem = bufs[(g + 1) % 2]
            pltpu.make_async_copy(
                src_hbm.at[pl.ds((g + 1) * ECH, ECH)], nsv, nsem).start()
            pltpu.make_async_copy(
                dst_hbm.at[pl.ds((g + 1) * ECH, ECH)], ndv, nsem).start()

        def grp(c, carry):
            s16 = sv[pl.ds(c * 16, 16)]
            d16 = dv[pl.ds(c * 16, 16)]
            mine = (s16 >= lo) & (s16 < lo + ROWS_PER_TILE)
            word = (s16 - lo) * WORDS + lax.shift_right_logical(d16, 5)
            word = jnp.where(mine, word, 0)
            bit = lax.shift_left(jnp.full((16,), 1, jnp.int32), d16 & 31)
            cnt = jnp.sum(mine.astype(jnp.int32))

            @pl.when(cnt > 0)
            def _():
                w = plsc.load_gather(bm, [word], mask=mine)
                plsc.store_scatter(bm, [word], w | bit, mask=mine)

                @pl.when(cnt > 1)
                def _():
                    def cond(pend):
                        return jnp.any(pend)

                    def body(pend):
                        w2 = plsc.load_gather(bm, [word], mask=pend)
                        miss = pend & ((w2 & bit) == 0)
                        plsc.store_scatter(bm, [word], w2 | bit, mask=miss)
                        return miss
                    lax.while_loop(cond, body, mine)
            return carry
        lax.fori_loop(0, ECH // 16, grp, 0)

    pltpu.sync_copy(bm, bm_hbm.at[pl.ds(wid * TILE_WORDS, TILE_WORDS)])


@functools.cache
def _bitmask_kernel():
    return functools.partial(
        pl.kernel,
        out_type=jax.ShapeDtypeStruct((N * WORDS,), jnp.int32),
        mesh=plsc.VectorSubcoreMesh(
            core_axis_name="c", subcore_axis_name="s",
            num_cores=NUM_CORES, num_subcores=NUM_SUBCORES,
        ),
        scratch_types=[
            pltpu.VMEM((ECH,), jnp.int32),
            pltpu.VMEM((ECH,), jnp.int32),
            pltpu.VMEM((ECH,), jnp.int32),
            pltpu.VMEM((ECH,), jnp.int32),
            pltpu.VMEM((TILE_WORDS,), jnp.int32),
            pltpu.SemaphoreType.DMA,
            pltpu.SemaphoreType.DMA,
        ],
        compiler_params=pltpu.CompilerParams(needs_layout_passes=False),
    )(_bitmask_body)


def _proj_body(nf_ref, wt_ref, b_ref, a1_ref, a2_ref, x_ref, s_ref, d_ref):
    xb = jnp.dot(nf_ref[...], wt_ref[...], preferred_element_type=jnp.float32)
    xb = xb + b_ref[...]
    x_ref[...] = xb
    s_ref[...] = jnp.dot(xb, a1_ref[...], preferred_element_type=jnp.float32)
    d_ref[...] = jnp.dot(xb, a2_ref[...], preferred_element_type=jnp.float32)


RP = 512
RB = 256


def _gat_body(s_ref, dt_ref, x_ref, m_ref, o_ref):
    s_blk = s_ref[...]
    mb = m_ref[...]
    rep = jnp.broadcast_to(mb[:, :, None], (RB, WORDS, 32)).reshape(RB, N)
    sh = lax.broadcasted_iota(jnp.int32, (RB, N), 1) & 31
    bitset = (lax.shift_right_logical(rep, sh) & 1) == 1
    for h in range(H):
        t = s_blk[:, h:h + 1] + dt_ref[h:h + 1, :]
        l = jnp.where(t > 0, t, ALPHA * t)
        l = jnp.where(bitset, l, jnp.float32(NEG))
        mx = jnp.max(l, axis=1, keepdims=True)
        e = jnp.exp(l - mx)
        den = jnp.sum(e, axis=1, keepdims=True)
        acc = jnp.dot(e, x_ref[:, h * CH:(h + 1) * CH],
                      preferred_element_type=jnp.float32)
        o_ref[:, h * CH:(h + 1) * CH] = acc / den


def kernel(node_feats, edge_index, W, b, a):
    src = edge_index[:, 0].astype(jnp.int32)
    dst = edge_index[:, 1].astype(jnp.int32)
    bm_flat = _bitmask_kernel()(src, dst)
    mask_bits = bm_flat.reshape(N, WORDS)

    eye = jnp.eye(H, dtype=jnp.float32)
    a1 = (a[:, :CH, None] * eye[:, None, :]).reshape(H * CH, H)
    a2 = (a[:, CH:, None] * eye[:, None, :]).reshape(H * CH, H)

    x, s, d = pl.pallas_call(
        _proj_body,
        grid=(N // RP,),
        in_specs=[
            pl.BlockSpec((RP, C_IN), lambda i: (i, 0)),
            pl.BlockSpec((C_IN, H * CH), lambda i: (0, 0)),
            pl.BlockSpec((1, H * CH), lambda i: (0, 0)),
            pl.BlockSpec((H * CH, H), lambda i: (0, 0)),
            pl.BlockSpec((H * CH, H), lambda i: (0, 0)),
        ],
        out_specs=[
            pl.BlockSpec((RP, H * CH), lambda i: (i, 0)),
            pl.BlockSpec((RP, H), lambda i: (i, 0)),
            pl.BlockSpec((RP, H), lambda i: (i, 0)),
        ],
        out_shape=[
            jax.ShapeDtypeStruct((N, H * CH), jnp.float32),
            jax.ShapeDtypeStruct((N, H), jnp.float32),
            jax.ShapeDtypeStruct((N, H), jnp.float32),
        ],
        compiler_params=pltpu.CompilerParams(
            dimension_semantics=("arbitrary",),
        ),
    )(node_feats, W.T, b.reshape(1, H * CH), a1, a2)

    out = pl.pallas_call(
        _gat_body,
        grid=(N // RB,),
        in_specs=[
            pl.BlockSpec((RB, H), lambda i: (i, 0)),
            pl.BlockSpec((H, N), lambda i: (0, 0)),
            pl.BlockSpec((N, H * CH), lambda i: (0, 0)),
            pl.BlockSpec((RB, WORDS), lambda i: (i, 0)),
        ],
        out_specs=pl.BlockSpec((RB, H * CH), lambda i: (i, 0)),
        out_shape=jax.ShapeDtypeStruct((N, H * CH), jnp.float32),
        compiler_params=pltpu.CompilerParams(
            dimension_semantics=("arbitrary",),
            vmem_limit_bytes=100 * 1024 * 1024,
        ),
    )(s, d.T, x, mask_bits)

    return out.reshape(1, N, H * CH)


# speedup vs baseline: 2.1227x; 2.1227x over previous
import functools

import jax
import jax.numpy as jnp
from jax import lax
from jax.experimental import pallas as pl
from jax.experimental.pallas import tpu as pltpu
from jax.experimental.pallas import tpu_sc as plsc

N = 4096
E = 131072
C_IN = 256
H = 4
CH = 64
ALPHA = 0.2
NEG = -9e15

NUM_CORES = 2
NUM_SUBCORES = 16
NW = NUM_CORES * NUM_SUBCORES
ROWS_PER_TILE = N // NW
WORDS = N // 32
TILE_WORDS = ROWS_PER_TILE * WORDS
ECH = 16384
NCH = E // ECH


def _bitmask_body(src_hbm, dst_hbm, bm_hbm,
                  src_a, dst_a, src_b, dst_b, bm, sem_a, sem_b):
    cid = lax.axis_index("c")
    sid = lax.axis_index("s")
    wid = cid * NUM_SUBCORES + sid
    lo = wid * ROWS_PER_TILE

    z16 = jnp.zeros((16,), jnp.int32)

    def zf(i, carry):
        bm[pl.ds(i * 16, 16)] = z16
        return carry
    lax.fori_loop(0, TILE_WORDS // 16, zf, 0)

    bufs = [(src_a, dst_a, sem_a), (src_b, dst_b, sem_b)]
    pltpu.make_async_copy(src_hbm.at[pl.ds(0, ECH)], src_a, sem_a).start()
    pltpu.make_async_copy(dst_hbm.at[pl.ds(0, ECH)], dst_a, sem_a).start()

    for g in range(NCH):
        sv, dv, sem = bufs[g % 2]
        pltpu.make_async_copy(src_hbm.at[pl.ds(g * ECH, ECH)], sv, sem).wait()
        pltpu.make_async_copy(dst_hbm.at[pl.ds(g * ECH, ECH)], dv, sem).wait()
        if g + 1 < NCH:
            nsv, ndv, nsem = bufs[(g + 1) % 2]
            pltpu.make_async_copy(
                src_hbm.at[pl.ds((g + 1) * ECH, ECH)], nsv, nsem).start()
            pltpu.make_async_copy(
                dst_hbm.at[pl.ds((g + 1) * ECH, ECH)], ndv, nsem).start()

        one16 = jnp.full((16,), 1, jnp.int32)

        def keys(c):
            s16 = sv[pl.ds(c * 16, 16)]
            d16 = dv[pl.ds(c * 16, 16)]
            mine = (s16 >= lo) & (s16 < lo + ROWS_PER_TILE)
            word = jnp.where(mine, (s16 - lo) * WORDS + (d16 & (WORDS - 1)), 0)
            bit = lax.shift_left(one16, lax.shift_right_logical(d16, 7))
            return mine, word, bit

        def grp(c, missacc):
            mine, word, bit = keys(c)
            w = plsc.load_gather(bm, [word], mask=mine)
            plsc.store_scatter(bm, [word], w | bit, mask=mine)
            w2 = plsc.load_gather(bm, [word], mask=mine)
            return missacc | (mine & ((w2 & bit) == 0))
        missacc = lax.fori_loop(0, ECH // 16, grp,
                                jnp.zeros((16,), jnp.bool_))

        @pl.when(jnp.any(missacc))
        def _():
            def rep(c, carry):
                mine, word, bit = keys(c)

                def cond(pend):
                    return jnp.any(pend)

                def body(pend):
                    w2 = plsc.load_gather(bm, [word], mask=pend)
                    miss = pend & ((w2 & bit) == 0)
                    plsc.store_scatter(bm, [word], w2 | bit, mask=miss)
                    return miss
                lax.while_loop(cond, body, mine)
                return carry
            lax.fori_loop(0, ECH // 16, rep, 0)

    pltpu.sync_copy(bm, bm_hbm.at[pl.ds(wid * TILE_WORDS, TILE_WORDS)])


@functools.cache
def _bitmask_kernel():
    return functools.partial(
        pl.kernel,
        out_type=jax.ShapeDtypeStruct((N * WORDS,), jnp.int32),
        mesh=plsc.VectorSubcoreMesh(
            core_axis_name="c", subcore_axis_name="s",
            num_cores=NUM_CORES, num_subcores=NUM_SUBCORES,
        ),
        scratch_types=[
            pltpu.VMEM((ECH,), jnp.int32),
            pltpu.VMEM((ECH,), jnp.int32),
            pltpu.VMEM((ECH,), jnp.int32),
            pltpu.VMEM((ECH,), jnp.int32),
            pltpu.VMEM((TILE_WORDS,), jnp.int32),
            pltpu.SemaphoreType.DMA,
            pltpu.SemaphoreType.DMA,
        ],
        compiler_params=pltpu.CompilerParams(needs_layout_passes=False),
    )(_bitmask_body)


def _proj_body(nf_ref, wt_ref, b_ref, a1_ref, a2_ref, x_ref, s_ref, d_ref):
    xb = jnp.dot(nf_ref[...], wt_ref[...], preferred_element_type=jnp.float32)
    xb = xb + b_ref[...]
    x_ref[...] = xb
    s_ref[...] = jnp.dot(xb, a1_ref[...], preferred_element_type=jnp.float32)
    d_ref[...] = jnp.dot(xb, a2_ref[...], preferred_element_type=jnp.float32)


RP = 512
RB = 256


def _gat_body(s_ref, dt_ref, x_ref, m_ref, o_ref):
    s_blk = s_ref[...]
    mb = m_ref[...]
    rep = jnp.concatenate([mb] * 32, axis=1)
    sh = lax.shift_right_logical(
        lax.broadcasted_iota(jnp.int32, (RB, N), 1), 7)
    bitset = (lax.shift_right_logical(rep, sh) & 1) == 1
    for h in range(H):
        t = s_blk[:, h:h + 1] + dt_ref[h:h + 1, :]
        l = jnp.where(t > 0, t, ALPHA * t)
        l = jnp.where(bitset, l, jnp.float32(NEG))
        mx = jnp.max(l, axis=1, keepdims=True)
        e = jnp.exp(l - mx)
        den = jnp.sum(e, axis=1, keepdims=True)
        acc = jnp.dot(e, x_ref[:, h * CH:(h + 1) * CH],
                      preferred_element_type=jnp.float32)
        o_ref[:, h * CH:(h + 1) * CH] = acc / den


def kernel(node_feats, edge_index, W, b, a):
    src = edge_index[:, 0].astype(jnp.int32)
    dst = edge_index[:, 1].astype(jnp.int32)
    bm_flat = _bitmask_kernel()(src, dst)
    mask_bits = bm_flat.reshape(N, WORDS)

    eye = jnp.eye(H, dtype=jnp.float32)
    a1 = (a[:, :CH, None] * eye[:, None, :]).reshape(H * CH, H)
    a2 = (a[:, CH:, None] * eye[:, None, :]).reshape(H * CH, H)

    x, s, d = pl.pallas_call(
        _proj_body,
        grid=(N // RP,),
        in_specs=[
            pl.BlockSpec((RP, C_IN), lambda i: (i, 0)),
            pl.BlockSpec((C_IN, H * CH), lambda i: (0, 0)),
            pl.BlockSpec((1, H * CH), lambda i: (0, 0)),
            pl.BlockSpec((H * CH, H), lambda i: (0, 0)),
            pl.BlockSpec((H * CH, H), lambda i: (0, 0)),
        ],
        out_specs=[
            pl.BlockSpec((RP, H * CH), lambda i: (i, 0)),
            pl.BlockSpec((RP, H), lambda i: (i, 0)),
            pl.BlockSpec((RP, H), lambda i: (i, 0)),
        ],
        out_shape=[
            jax.ShapeDtypeStruct((N, H * CH), jnp.float32),
            jax.ShapeDtypeStruct((N, H), jnp.float32),
            jax.ShapeDtypeStruct((N, H), jnp.float32),
        ],
        compiler_params=pltpu.CompilerParams(
            dimension_semantics=("arbitrary",),
        ),
    )(node_feats, W.T, b.reshape(1, H * CH), a1, a2)

    out = pl.pallas_call(
        _gat_body,
        grid=(N // RB,),
        in_specs=[
            pl.BlockSpec((RB, H), lambda i: (i, 0)),
            pl.BlockSpec((H, N), lambda i: (0, 0)),
            pl.BlockSpec((N, H * CH), lambda i: (0, 0)),
            pl.BlockSpec((RB, WORDS), lambda i: (i, 0)),
        ],
        out_specs=pl.BlockSpec((RB, H * CH), lambda i: (i, 0)),
        out_shape=jax.ShapeDtypeStruct((N, H * CH), jnp.float32),
        compiler_params=pltpu.CompilerParams(
            dimension_semantics=("arbitrary",),
            vmem_limit_bytes=100 * 1024 * 1024,
        ),
    )(s, d.T, x, mask_bits)

    return out.reshape(1, N, H * CH)


# speedup vs baseline: 2.2192x; 1.0454x over previous
import functools

import jax
import jax.numpy as jnp
from jax import lax
from jax.experimental import pallas as pl
from jax.experimental.pallas import tpu as pltpu
from jax.experimental.pallas import tpu_sc as plsc

N = 4096
E = 131072
C_IN = 256
H = 4
CH = 64
ALPHA = 0.2
NEG = -9e15

NUM_CORES = 2
NUM_SUBCORES = 16
NW = NUM_CORES * NUM_SUBCORES
ROWS_PER_TILE = N // NW
WORDS = N // 32
TILE_WORDS = ROWS_PER_TILE * WORDS
ECH = 16384
NCH = E // ECH


def _bitmask_body(src_hbm, dst_hbm, bm_hbm,
                  src_a, dst_a, src_b, dst_b, bm, sem_a, sem_b):
    cid = lax.axis_index("c")
    sid = lax.axis_index("s")
    wid = cid * NUM_SUBCORES + sid
    lo = wid * ROWS_PER_TILE

    z16 = jnp.zeros((16,), jnp.int32)

    def zf(i, carry):
        bm[pl.ds(i * 16, 16)] = z16
        return carry
    lax.fori_loop(0, TILE_WORDS // 16, zf, 0)

    bufs = [(src_a, dst_a, sem_a), (src_b, dst_b, sem_b)]
    pltpu.make_async_copy(src_hbm.at[pl.ds(0, ECH)], src_a, sem_a).start()
    pltpu.make_async_copy(dst_hbm.at[pl.ds(0, ECH)], dst_a, sem_a).start()

    for g in range(NCH):
        sv, dv, sem = bufs[g % 2]
        pltpu.make_async_copy(src_hbm.at[pl.ds(g * ECH, ECH)], sv, sem).wait()
        pltpu.make_async_copy(dst_hbm.at[pl.ds(g * ECH, ECH)], dv, sem).wait()
        if g + 1 < NCH:
            nsv, ndv, nsem = bufs[(g + 1) % 2]
            pltpu.make_async_copy(
                src_hbm.at[pl.ds((g + 1) * ECH, ECH)], nsv, nsem).start()
            pltpu.make_async_copy(
                dst_hbm.at[pl.ds((g + 1) * ECH, ECH)], ndv, nsem).start()

        one16 = jnp.full((16,), 1, jnp.int32)

        def keys(c):
            s16 = sv[pl.ds(c * 16, 16)]
            d16 = dv[pl.ds(c * 16, 16)]
            mine = (s16 >= lo) & (s16 < lo + ROWS_PER_TILE)
            word = jnp.where(mine, (s16 - lo) * WORDS + (d16 & (WORDS - 1)), 0)
            bit = lax.shift_left(one16, lax.shift_right_logical(d16, 7))
            return mine, word, bit

        def grp(c, missacc):
            mine, word, bit = keys(c)
            w = plsc.load_gather(bm, [word], mask=mine)
            plsc.store_scatter(bm, [word], w | bit, mask=mine)
            w2 = plsc.load_gather(bm, [word], mask=mine)
            return missacc | (mine & ((w2 & bit) == 0))
        missacc = lax.fori_loop(0, ECH // 16, grp,
                                jnp.zeros((16,), jnp.bool_))

        @pl.when(jnp.any(missacc))
        def _():
            def rep(c, carry):
                mine, word, bit = keys(c)

                def cond(pend):
                    return jnp.any(pend)

                def body(pend):
                    w2 = plsc.load_gather(bm, [word], mask=pend)
                    miss = pend & ((w2 & bit) == 0)
                    plsc.store_scatter(bm, [word], w2 | bit, mask=miss)
                    return miss
                lax.while_loop(cond, body, mine)
                return carry
            lax.fori_loop(0, ECH // 16, rep, 0)

    pltpu.sync_copy(bm, bm_hbm.at[pl.ds(wid * TILE_WORDS, TILE_WORDS)])


@functools.cache
def _bitmask_kernel():
    return functools.partial(
        pl.kernel,
        out_type=jax.ShapeDtypeStruct((N * WORDS,), jnp.int32),
        mesh=plsc.VectorSubcoreMesh(
            core_axis_name="c", subcore_axis_name="s",
            num_cores=NUM_CORES, num_subcores=NUM_SUBCORES,
        ),
        scratch_types=[
            pltpu.VMEM((ECH,), jnp.int32),
            pltpu.VMEM((ECH,), jnp.int32),
            pltpu.VMEM((ECH,), jnp.int32),
            pltpu.VMEM((ECH,), jnp.int32),
            pltpu.VMEM((TILE_WORDS,), jnp.int32),
            pltpu.SemaphoreType.DMA,
            pltpu.SemaphoreType.DMA,
        ],
        compiler_params=pltpu.CompilerParams(needs_layout_passes=False),
    )(_bitmask_body)


def _proj_body(nf_ref, wt_ref, b_ref, a1_ref, a2_ref,
               x_ref, s_ref, d_ref, cs_ref, dm_ref):
    xb = jnp.dot(nf_ref[...], wt_ref[...], preferred_element_type=jnp.float32)
    xb = xb + b_ref[...]
    x_ref[...] = xb
    s_ref[...] = jnp.dot(xb, a1_ref[...], preferred_element_type=jnp.float32)
    db = jnp.dot(xb, a2_ref[...], preferred_element_type=jnp.float32)
    d_ref[...] = db

    @pl.when(pl.program_id(0) == 0)
    def _():
        cs_ref[...] = jnp.zeros_like(cs_ref)
        dm_ref[...] = jnp.full_like(dm_ref, -jnp.inf)
    cs_ref[...] += jnp.sum(xb, axis=0, keepdims=True)
    dm_ref[...] = jnp.maximum(dm_ref[...], jnp.max(db, axis=0, keepdims=True))


RP = 512
RB = 256


def _gat_body(s_ref, dt_ref, x_ref, m_ref, mean_ref, dm_ref, o_ref):
    s_blk = s_ref[...]
    mb = m_ref[...]
    rep = jnp.concatenate([mb] * 32, axis=1)
    sh = lax.shift_right_logical(
        lax.broadcasted_iota(jnp.int32, (RB, N), 1), 7)
    bitset = (lax.shift_right_logical(rep, sh) & 1) == 1
    sD = s_blk + dm_ref[...]
    M = jnp.where(sD > 0, sD, ALPHA * sD)
    for h in range(H):
        Mh = M[:, h:h + 1]
        c2 = (ALPHA - 1.0) * Mh
        t2 = (s_blk[:, h:h + 1] - Mh) + dt_ref[h:h + 1, :]
        arg = jnp.where(t2 > -Mh, t2, ALPHA * t2 + c2)
        arg = jnp.where(bitset, arg, jnp.float32(NEG))
        e = jnp.exp(arg)
        den = jnp.sum(e, axis=1, keepdims=True)
        acc = jnp.dot(e, x_ref[:, h * CH:(h + 1) * CH],
                      preferred_element_type=jnp.float32)
        o_ref[:, h * CH:(h + 1) * CH] = jnp.where(
            den == 0.0, mean_ref[:, h * CH:(h + 1) * CH], acc / den)


def kernel(node_feats, edge_index, W, b, a):
    src = edge_index[:, 0].astype(jnp.int32)
    dst = edge_index[:, 1].astype(jnp.int32)
    bm_flat = _bitmask_kernel()(src, dst)
    mask_bits = bm_flat.reshape(N, WORDS)

    eye = jnp.eye(H, dtype=jnp.float32)
    a1 = (a[:, :CH, None] * eye[:, None, :]).reshape(H * CH, H)
    a2 = (a[:, CH:, None] * eye[:, None, :]).reshape(H * CH, H)

    x, s, d, cs, dm = pl.pallas_call(
        _proj_body,
        grid=(N // RP,),
        in_specs=[
            pl.BlockSpec((RP, C_IN), lambda i: (i, 0)),
            pl.BlockSpec((C_IN, H * CH), lambda i: (0, 0)),
            pl.BlockSpec((1, H * CH), lambda i: (0, 0)),
            pl.BlockSpec((H * CH, H), lambda i: (0, 0)),
            pl.BlockSpec((H * CH, H), lambda i: (0, 0)),
        ],
        out_specs=[
            pl.BlockSpec((RP, H * CH), lambda i: (i, 0)),
            pl.BlockSpec((RP, H), lambda i: (i, 0)),
            pl.BlockSpec((RP, H), lambda i: (i, 0)),
            pl.BlockSpec((1, H * CH), lambda i: (0, 0)),
            pl.BlockSpec((1, H), lambda i: (0, 0)),
        ],
        out_shape=[
            jax.ShapeDtypeStruct((N, H * CH), jnp.float32),
            jax.ShapeDtypeStruct((N, H), jnp.float32),
            jax.ShapeDtypeStruct((N, H), jnp.float32),
            jax.ShapeDtypeStruct((1, H * CH), jnp.float32),
            jax.ShapeDtypeStruct((1, H), jnp.float32),
        ],
        compiler_params=pltpu.CompilerParams(
            dimension_semantics=("arbitrary",),
        ),
    )(node_feats, W.T, b.reshape(1, H * CH), a1, a2)
    mean = cs * (1.0 / N)

    out = pl.pallas_call(
        _gat_body,
        grid=(N // RB,),
        in_specs=[
            pl.BlockSpec((RB, H), lambda i: (i, 0)),
            pl.BlockSpec((H, N), lambda i: (0, 0)),
            pl.BlockSpec((N, H * CH), lambda i: (0, 0)),
            pl.BlockSpec((RB, WORDS), lambda i: (i, 0)),
            pl.BlockSpec((1, H * CH), lambda i: (0, 0)),
            pl.BlockSpec((1, H), lambda i: (0, 0)),
        ],
        out_specs=pl.BlockSpec((RB, H * CH), lambda i: (i, 0)),
        out_shape=jax.ShapeDtypeStruct((N, H * CH), jnp.float32),
        compiler_params=pltpu.CompilerParams(
            dimension_semantics=("arbitrary",),
            vmem_limit_bytes=100 * 1024 * 1024,
        ),
    )(s, d.T, x, mask_bits, mean, dm)

    return out.reshape(1, N, H * CH)
